# baseline (speedup 1.0000x reference)
import functools

import jax
import jax.numpy as jnp
import numpy as np
from jax import lax
from jax.experimental import pallas as pl
from jax.experimental.pallas import tpu as pltpu
from jax.experimental.pallas import tpu_sc as plsc

_TIME_STEPS = 1000
_BETA_START = 0.0001
_BETA_END = 0.02

_betas = np.linspace(_BETA_START, _BETA_END, _TIME_STEPS, dtype=np.float64)
_alphas_cumprod = np.cumprod(1.0 - _betas)
_TABLE2D = np.zeros((_TIME_STEPS, 128), dtype=np.float32)
_TABLE2D[:, 0] = np.sqrt(_alphas_cumprod)
_TABLE2D[:, 1] = np.sqrt(1.0 - _alphas_cumprod)

_BB = 16


def _make_sc_gather(batch):
    info = plsc.get_sparse_core_info()
    nc, ns = info.num_cores, info.num_subcores
    nw = nc * ns
    b_per_w = batch // nw
    mesh = plsc.VectorSubcoreMesh(core_axis_name="c", subcore_axis_name="s")

    @functools.partial(
        pl.kernel,
        mesh=mesh,
        out_type=jax.ShapeDtypeStruct((batch, 128), jnp.float32),
        scratch_types=[
            pltpu.VMEM((b_per_w,), jnp.int32),
            pltpu.VMEM((b_per_w, 128), jnp.float32),
            pltpu.SemaphoreType.DMA,
        ],
    )
    def gather_k(tab_hbm, idx_hbm, out_hbm, idx_v, rows_v, sem):
        wid = lax.axis_index("s") * nc + lax.axis_index("c")
        base = wid * b_per_w
        pltpu.sync_copy(idx_hbm.at[pl.ds(base, b_per_w)], idx_v)
        pltpu.async_copy(tab_hbm.at[idx_v], rows_v, sem).wait()
        pltpu.sync_copy(rows_v, out_hbm.at[pl.ds(base, b_per_w)])

    return gather_k


def _fma_body(scales_ref, x_ref, n_ref, o_ref):
    g = pl.program_id(0)
    for i in range(_BB):
        a = scales_ref[g * _BB + i, 0]
        c = scales_ref[g * _BB + i, 1]
        o_ref[i] = a * x_ref[i] + c * n_ref[i]


@jax.jit
def kernel(x, time, noise):
    b, ch, h, w = x.shape
    rows = ch * h * w // 128
    x3 = x.reshape(b, rows, 128)
    n3 = noise.reshape(b, rows, 128)
    tab = jnp.asarray(_TABLE2D)

    scales = _make_sc_gather(b)(tab, time)[:, :2]

    grid = b // _BB
    spec = pl.BlockSpec((_BB, rows, 128), lambda g, *_: (g, 0, 0))
    out = pl.pallas_call(
        _fma_body,
        grid_spec=pltpu.PrefetchScalarGridSpec(
            num_scalar_prefetch=1,
            grid=(grid,),
            in_specs=[spec, spec],
            out_specs=spec,
        ),
        out_shape=jax.ShapeDtypeStruct((b, rows, 128), jnp.float32),
    )(scales, x3, n3)
    return out.reshape(x.shape)


# speedup vs baseline: 1.0372x; 1.0372x over previous
import functools

import jax
import jax.numpy as jnp
import numpy as np
from jax import lax
from jax.experimental import pallas as pl
from jax.experimental.pallas import tpu as pltpu
from jax.experimental.pallas import tpu_sc as plsc

_TIME_STEPS = 1000
_BETA_START = 0.0001
_BETA_END = 0.02

_betas = np.linspace(_BETA_START, _BETA_END, _TIME_STEPS, dtype=np.float64)
_alphas_cumprod = np.cumprod(1.0 - _betas)
_TABLE2D = np.zeros((_TIME_STEPS, 128), dtype=np.float32)
_TABLE2D[:, 0] = np.sqrt(_alphas_cumprod)
_TABLE2D[:, 1] = np.sqrt(1.0 - _alphas_cumprod)

_BB = 16


def _make_sc_gather(batch):
    info = plsc.get_sparse_core_info()
    nc, ns = info.num_cores, info.num_subcores
    nw = nc * ns
    b_per_w = batch // nw
    mesh = plsc.VectorSubcoreMesh(core_axis_name="c", subcore_axis_name="s")

    @functools.partial(
        pl.kernel,
        mesh=mesh,
        out_type=jax.ShapeDtypeStruct((batch, 128), jnp.float32),
        scratch_types=[
            pltpu.VMEM((b_per_w,), jnp.int32),
            pltpu.VMEM((b_per_w, 128), jnp.float32),
            pltpu.SemaphoreType.DMA,
        ],
    )
    def gather_k(tab_hbm, idx_hbm, out_hbm, idx_v, rows_v, sem):
        wid = lax.axis_index("s") * nc + lax.axis_index("c")
        base = wid * b_per_w
        pltpu.sync_copy(idx_hbm.at[pl.ds(base, b_per_w)], idx_v)
        pltpu.async_copy(tab_hbm.at[idx_v], rows_v, sem).wait()
        pltpu.sync_copy(rows_v, out_hbm.at[pl.ds(base, b_per_w)])

    return gather_k


def _fma_body(scales_ref, x_ref, n_ref, o_ref):
    for i in range(_BB):
        a = scales_ref[pl.ds(i, 1), pl.ds(0, 1)]
        c = scales_ref[pl.ds(i, 1), pl.ds(1, 1)]
        o_ref[i] = a * x_ref[i] + c * n_ref[i]


@jax.jit
def kernel(x, time, noise):
    b, ch, h, w = x.shape
    rows = ch * h * w // 128
    x3 = x.reshape(b, rows, 128)
    n3 = noise.reshape(b, rows, 128)
    tab = jnp.asarray(_TABLE2D)

    scales = _make_sc_gather(b)(tab, time)

    grid = b // _BB
    spec = pl.BlockSpec((_BB, rows, 128), lambda g: (g, 0, 0))
    sspec = pl.BlockSpec((_BB, 128), lambda g: (g, 0))
    out = pl.pallas_call(
        _fma_body,
        grid=(grid,),
        in_specs=[sspec, spec, spec],
        out_specs=spec,
        out_shape=jax.ShapeDtypeStruct((b, rows, 128), jnp.float32),
    )(scales, x3, n3)
    return out.reshape(x.shape)
